# Initial kernel scaffold; baseline (speedup 1.0000x reference)
import jax
import jax.numpy as jnp
from jax.experimental import pallas as pl


def kernel(x, edge_index, batch_indeces, gcn_W, gcn_b, gcn_gamma, gcn_beta, mu_W, mu_b, mu_gamma, mu_beta, ls_W, ls_b, ls_gamma, ls_beta):
    raise NotImplementedError("write your pallas kernel here")


# speedup vs baseline: 12.2965x; 12.2965x over previous
import functools

import jax
import jax.numpy as jnp
from jax import lax
from jax.experimental import pallas as pl
from jax.experimental.pallas import tpu as pltpu
from jax.experimental.pallas import tpu_sc as plsc

NN = 10000
NP = 10240
EE = 320000
GG = 64
BR = 512
NB = NP // BR
BN_EPS = 1e-5
MAXLOGSTD = 10.0

NCORES = 2
NSUB = 16
NTILES = NCORES * NSUB
CH = 128
EPT = EE // NTILES
NCHUNK = -(-EPT // CH)
EPAD = NTILES * NCHUNK * CH
RPS = NP // NSUB

@functools.cache
def _get_mesh():
    return plsc.VectorSubcoreMesh(core_axis_name="c", subcore_axis_name="s")


@functools.cache
def _get_sc_deg():
    @functools.partial(
        pl.kernel,
        out_type=jax.ShapeDtypeStruct((NCORES, NP), jnp.float32),
        mesh=_get_mesh(),
        scratch_types=[
            pltpu.VMEM((NCHUNK, CH), jnp.int32),
            pltpu.VMEM((CH,), jnp.float32),
            pltpu.VMEM((RPS,), jnp.float32),
            pltpu.VMEM_SHARED((NP,), jnp.float32),
        ],
    )
    def _sc_deg(dst3, deg_out, idx_v, ones_v, zv, acc):
        c = lax.axis_index("c")
        s = lax.axis_index("s")
        wid = c * NSUB + s

        def fill_ones(i, carry):
            ones_v[pl.ds(i * 16, 16)] = jnp.full((16,), 1.0, jnp.float32)
            return carry

        lax.fori_loop(0, CH // 16, fill_ones, 0)

        def fill_zero(i, carry):
            zv[pl.ds(i * 16, 16)] = jnp.zeros((16,), jnp.float32)
            return carry

        lax.fori_loop(0, RPS // 16, fill_zero, 0)
        pltpu.sync_copy(zv, acc.at[pl.ds(s * RPS, RPS)])
        plsc.subcore_barrier()

        pltpu.sync_copy(dst3.at[wid], idx_v)

        def scat(j, carry):
            pltpu.sync_copy(ones_v, acc.at[idx_v.at[j]], add=True)
            return carry

        lax.fori_loop(0, NCHUNK, scat, 0)
        plsc.subcore_barrier()
        pltpu.sync_copy(
            acc.at[pl.ds(s * RPS, RPS)], deg_out.at[c, pl.ds(s * RPS, RPS)]
        )

    return _sc_deg


@functools.cache
def _get_sc_scatter(dout):
    @functools.partial(
        pl.kernel,
        out_type=jax.ShapeDtypeStruct((NCORES, NP, dout), jnp.float32),
        mesh=_get_mesh(),
        scratch_types=[
            pltpu.VMEM((NCHUNK, CH), jnp.int32),
            pltpu.VMEM((NCHUNK, CH), jnp.int32),
            pltpu.VMEM((CH, dout), jnp.float32),
            pltpu.VMEM_SHARED((NP, dout), jnp.float32),
            pltpu.SemaphoreType.DMA,
        ],
        compiler_params=pltpu.CompilerParams(use_tc_tiling_on_sc=False),
    )
    def sc_scatter(tmp_hbm, src3, dst3, part_out, src_v, dst_v, rows_v, acc, sem):
        c = lax.axis_index("c")
        s = lax.axis_index("s")
        wid = c * NSUB + s
        kk = dout // 16

        def fill_zero(i, carry):
            rows_v[i // kk, pl.ds((i % kk) * 16, 16)] = jnp.zeros((16,), jnp.float32)
            return carry

        lax.fori_loop(0, CH * kk, fill_zero, 0)

        def zero_acc(t, carry):
            pltpu.sync_copy(rows_v, acc.at[pl.ds(s * RPS + t * CH, CH)])
            return carry

        lax.fori_loop(0, RPS // CH, zero_acc, 0)
        plsc.subcore_barrier()

        pltpu.sync_copy(src3.at[wid], src_v)
        pltpu.sync_copy(dst3.at[wid], dst_v)

        def body(j, carry):
            pltpu.async_copy(tmp_hbm.at[src_v.at[j]], rows_v, sem).wait()
            pltpu.sync_copy(rows_v, acc.at[dst_v.at[j]], add=True)
            return carry

        lax.fori_loop(0, NCHUNK, body, 0)
        plsc.subcore_barrier()

        def copy_out(t, carry):
            sl = pl.ds(s * RPS + t * CH, CH)
            pltpu.sync_copy(acc.at[sl], part_out.at[c, sl])
            return carry

        lax.fori_loop(0, RPS // CH, copy_out, 0)

    return sc_scatter


def _entry_call(deg3, x_p, W1):
    dout = W1.shape[1]

    def body(deg_ref, x_ref, w_ref, tmp_ref, dinv_ref):
        deg = deg_ref[0] + deg_ref[1]
        dinv = lax.rsqrt(1.0 + deg)
        hw = jnp.dot(x_ref[...], w_ref[...], preferred_element_type=jnp.float32)
        tmp_ref[...] = hw * dinv
        dinv_ref[...] = dinv

    return pl.pallas_call(
        body,
        grid=(NB,),
        in_specs=[
            pl.BlockSpec((2, BR, 1), lambda i: (0, i, 0)),
            pl.BlockSpec((BR, x_p.shape[1]), lambda i: (i, 0)),
            pl.BlockSpec(W1.shape, lambda i: (0, 0)),
        ],
        out_specs=[
            pl.BlockSpec((BR, dout), lambda i: (i, 0)),
            pl.BlockSpec((BR, 1), lambda i: (i, 0)),
        ],
        out_shape=[
            jax.ShapeDtypeStruct((NP, dout), jnp.float32),
            jax.ShapeDtypeStruct((NP, 1), jnp.float32),
        ],
    )(deg3, x_p, W1)


def _finish_call(part, tmp, dinv, b, g, bt, Wn):
    dout = tmp.shape[1]
    last = Wn is None
    dnext = dout if last else Wn.shape[1]

    def body(p_ref, tmp_ref, dinv_ref, b_ref, g_ref, bt_ref, w_ref, out_ref, h_s, stat_s):
        ph = pl.program_id(0)
        i = pl.program_id(1)

        @pl.when(ph == 0)
        def _():
            h = (p_ref[0] + p_ref[1] + tmp_ref[...]) * dinv_ref[...] + b_ref[...]
            rid = lax.broadcasted_iota(jnp.int32, (BR, 1), 0) + i * BR
            hm = jnp.where(rid < NN, h, 0.0)

            @pl.when(i == 0)
            def _():
                stat_s[...] = jnp.zeros_like(stat_s)

            stat_s[0:1, :] += jnp.sum(hm, axis=0, keepdims=True)
            h_s[pl.ds(i * BR, BR), :] = h

        @pl.when(ph == 1)
        def _():
            m = stat_s[0:1, :] / NN
            h = h_s[pl.ds(i * BR, BR), :]
            rid = lax.broadcasted_iota(jnp.int32, (BR, 1), 0) + i * BR
            hc = jnp.where(rid < NN, h - m, 0.0)
            stat_s[1:2, :] += jnp.sum(hc * hc, axis=0, keepdims=True)

        @pl.when(ph == 2)
        def _():
            m = stat_s[0:1, :] / NN
            var = stat_s[1:2, :] / NN
            h = h_s[pl.ds(i * BR, BR), :]
            hn = (h - m) * lax.rsqrt(var + BN_EPS) * g_ref[...] + bt_ref[...]
            hn = jnp.maximum(hn, 0.0)
            if last:
                out_ref[...] = hn
            else:
                out_ref[...] = (
                    jnp.dot(hn, w_ref[...], preferred_element_type=jnp.float32)
                    * dinv_ref[...]
                )

    wn = jnp.zeros((dout, dnext), jnp.float32) if last else Wn
    return pl.pallas_call(
        body,
        grid=(3, NB),
        in_specs=[
            pl.BlockSpec((2, BR, dout), lambda p, i: (0, i, 0)),
            pl.BlockSpec((BR, dout), lambda p, i: (i, 0)),
            pl.BlockSpec((BR, 1), lambda p, i: (i, 0)),
            pl.BlockSpec((1, dout), lambda p, i: (0, 0)),
            pl.BlockSpec((1, dout), lambda p, i: (0, 0)),
            pl.BlockSpec((1, dout), lambda p, i: (0, 0)),
            pl.BlockSpec((dout, dnext), lambda p, i: (0, 0)),
        ],
        out_specs=pl.BlockSpec((BR, dnext), lambda p, i: (i, 0)),
        out_shape=jax.ShapeDtypeStruct((NP, dnext), jnp.float32),
        scratch_shapes=[
            pltpu.VMEM((NP, dout), jnp.float32),
            pltpu.VMEM((8, dout), jnp.float32),
        ],
    )(part, tmp, dinv, b.reshape(1, dout), g.reshape(1, dout), bt.reshape(1, dout), wn)


def _pool_call(h5, batch_p):
    F = h5.shape[1]

    def body(h_ref, b_ref, zt_ref, z_s):
        i = pl.program_id(0)

        @pl.when(i == 0)
        def _():
            z_s[...] = jnp.full((F, GG), -jnp.inf, jnp.float32)

        onehot = b_ref[...] == lax.broadcasted_iota(jnp.int32, (BR, GG), 1)
        h = h_ref[...]
        for f in range(F):
            masked = jnp.where(onehot, h[:, f : f + 1], -jnp.inf)
            z_s[f : f + 1, :] = jnp.maximum(
                z_s[f : f + 1, :], jnp.max(masked, axis=0, keepdims=True)
            )

        @pl.when(i == NB - 1)
        def _():
            zt_ref[...] = z_s[...]

    return pl.pallas_call(
        body,
        grid=(NB,),
        in_specs=[
            pl.BlockSpec((BR, F), lambda i: (i, 0)),
            pl.BlockSpec((BR, 1), lambda i: (i, 0)),
        ],
        out_specs=pl.BlockSpec((F, GG), lambda i: (0, 0)),
        out_shape=jax.ShapeDtypeStruct((F, GG), jnp.float32),
        scratch_shapes=[pltpu.VMEM((F, GG), jnp.float32)],
    )(h5, batch_p)


def _heads_call(z, mu_W, mu_b, mu_g, mu_bt, ls_W, ls_b, ls_g, ls_bt, eps):
    L = len(mu_W)
    n_out = mu_W[-1].shape[1]

    def mlp(h, Ws, bs, gs, bts):
        for i in range(L):
            h = jnp.dot(h, Ws[i], preferred_element_type=jnp.float32) + bs[i]
            if i < L - 1:
                m = jnp.mean(h, axis=0, keepdims=True)
                v = jnp.mean((h - m) ** 2, axis=0, keepdims=True)
                h = (h - m) * lax.rsqrt(v + BN_EPS) * gs[i] + bts[i]
                h = jnp.maximum(h, 0.0)
        return h

    def body(*refs):
        vals = [r[...] for r in refs[:-1]]
        out_ref = refs[-1]
        zz = vals[0]
        idx = 1
        groups = []
        for _ in range(2):
            Ws = vals[idx : idx + L]; idx += L
            bs = vals[idx : idx + L]; idx += L
            gs = vals[idx : idx + L - 1]; idx += L - 1
            bts = vals[idx : idx + L - 1]; idx += L - 1
            groups.append((Ws, bs, gs, bts))
        ev = vals[idx]
        mu = mlp(zz, *groups[0])
        ls = jnp.minimum(mlp(zz, *groups[1]), MAXLOGSTD)
        out_ref[...] = mu + ev * jnp.exp(ls)

    args = (
        [z]
        + list(mu_W) + [b.reshape(1, -1) for b in mu_b]
        + [g.reshape(1, -1) for g in mu_g] + [b.reshape(1, -1) for b in mu_bt]
        + list(ls_W) + [b.reshape(1, -1) for b in ls_b]
        + [g.reshape(1, -1) for g in ls_g] + [b.reshape(1, -1) for b in ls_bt]
        + [eps]
    )
    return pl.pallas_call(
        body,
        out_shape=jax.ShapeDtypeStruct((GG, n_out), jnp.float32),
    )(*args)


def kernel(x, edge_index, batch_indeces, gcn_W, gcn_b, gcn_gamma, gcn_beta,
           mu_W, mu_b, mu_gamma, mu_beta, ls_W, ls_b, ls_gamma, ls_beta):
    x_p = jnp.pad(x, ((0, NP - NN), (0, 0)))
    src = edge_index[0]
    dst = edge_index[1]
    epad = jnp.full((EPAD - EE,), NN, jnp.int32)
    src3 = jnp.concatenate([src, epad]).reshape(NTILES, NCHUNK, CH)
    dst3 = jnp.concatenate([dst, epad]).reshape(NTILES, NCHUNK, CH)
    batch_p = jnp.concatenate(
        [batch_indeces, jnp.full((NP - NN,), GG, jnp.int32)]
    ).reshape(NP, 1)
    n_out = mu_W[-1].shape[1]
    eps = jax.random.normal(jax.random.key(42), (GG, n_out), jnp.float32)

    degp = _get_sc_deg()(dst3)
    deg3 = degp.reshape(NCORES, NP, 1)
    tmp, dinv = _entry_call(deg3, x_p, gcn_W[0])

    L = len(gcn_W)
    h5 = None
    for l in range(L):
        dout = gcn_W[l].shape[1]
        part = _get_sc_scatter(dout)(tmp, src3, dst3)
        wn = gcn_W[l + 1] if l + 1 < L else None
        res = _finish_call(part, tmp, dinv, gcn_b[l], gcn_gamma[l], gcn_beta[l], wn)
        if l + 1 < L:
            tmp = res
        else:
            h5 = res

    zt = _pool_call(h5, batch_p)
    z = zt.T[:GG]
    out = _heads_call(z, mu_W, mu_b, mu_gamma, mu_beta,
                      ls_W, ls_b, ls_gamma, ls_beta, eps)
    return (out, z)
